# Initial kernel scaffold; baseline (speedup 1.0000x reference)
import jax
import jax.numpy as jnp
from jax.experimental import pallas as pl


def kernel(input, span_idxs, W, b):
    raise NotImplementedError("write your pallas kernel here")


# speedup vs baseline: 1.0019x; 1.0019x over previous
import functools

import jax
import jax.numpy as jnp
from jax import lax
from jax.experimental import pallas as pl
from jax.experimental.pallas import tpu as pltpu
from jax.experimental.pallas import tpu_sc as plsc

B, S, L, V, D = 4096, 16, 200, 1000, 128
VP = 1024
BM = 512


def _mm_body(bow_ref, wt_ref, b_ref, o_ref):
    o_ref[...] = (
        jnp.dot(bow_ref[...], wt_ref[...], preferred_element_type=jnp.float32)
        + b_ref[...]
    )


def _matmul(bow, wt, b2):
    R = bow.shape[0]
    return pl.pallas_call(
        _mm_body,
        grid=(R // BM,),
        in_specs=[
            pl.BlockSpec((BM, VP), lambda i: (i, 0)),
            pl.BlockSpec((VP, D), lambda i: (0, 0)),
            pl.BlockSpec((1, D), lambda i: (0, 0)),
        ],
        out_specs=pl.BlockSpec((BM, D), lambda i: (i, 0)),
        out_shape=jax.ShapeDtypeStruct((R, D), jnp.float32),
    )(bow, wt, b2)


def _build_bow_xla(input, span_idxs):
    positions = jnp.arange(L)
    lo = span_idxs[..., 0:1]
    hi = span_idxs[..., 1:2]
    mask = (positions[None, None, :] >= lo) & (positions[None, None, :] < hi)
    b_idx = jnp.arange(B)[:, None, None]
    s_idx = jnp.arange(S)[None, :, None]
    tok = jnp.broadcast_to(input[:, None, :], (B, S, L))
    bow = jnp.zeros((B, S, VP), dtype=jnp.float32).at[b_idx, s_idx, tok].max(
        mask.astype(jnp.float32)
    )
    return bow.reshape(B * S, VP)


def kernel(input, span_idxs, W, b):
    bow = _build_bow_xla(input, span_idxs)
    wt = jnp.zeros((VP, D), jnp.float32).at[:V].set(W.T)
    out = _matmul(bow, wt, b.reshape(1, D))
    return out.reshape(B, S, D)


# speedup vs baseline: 56.3357x; 56.2305x over previous
import functools

import jax
import jax.numpy as jnp
from jax import lax
from jax.experimental import pallas as pl
from jax.experimental.pallas import tpu as pltpu
from jax.experimental.pallas import tpu_sc as plsc

B, S, L, V, D = 4096, 16, 200, 1000, 128
VP = 1024
BM = 512

NC, NS, LANES = 2, 16, 16
NW = NC * NS
BPW = B // NW
OFFS = list(range(0, L - LANES, LANES)) + [L - LANES]

_sc_mesh = plsc.VectorSubcoreMesh(core_axis_name="c", subcore_axis_name="s")


@functools.partial(
    pl.kernel,
    mesh=_sc_mesh,
    compiler_params=pltpu.CompilerParams(needs_layout_passes=False),
    out_type=jax.ShapeDtypeStruct((B * S * VP,), jnp.float32),
    scratch_types=[
        pltpu.VMEM((L,), jnp.int32),
        pltpu.VMEM((2 * S,), jnp.int32),
        pltpu.VMEM((S * VP,), jnp.float32),
        pltpu.VMEM((S * VP,), jnp.float32),
        pltpu.SemaphoreType.DMA,
        pltpu.SemaphoreType.DMA,
    ],
)
def _sc_bow(inp_hbm, spans_hbm, bow_hbm, tok_v, span_v, rows0, rows1, sem0, sem1):
    ci = lax.axis_index("c")
    si = lax.axis_index("s")
    wid = si * NC + ci
    base_b = wid * BPW

    ones = jnp.ones((LANES,), jnp.float32)
    zeros = jnp.zeros((LANES,), jnp.float32)
    iota16 = lax.iota(jnp.int32, LANES)

    def do_batch(bi, rows, sem, do_wait):
        @pl.when(do_wait)
        def _():
            pltpu.make_async_copy(rows, bow_hbm.at[pl.ds(0, S * VP)], sem).wait()

        pltpu.sync_copy(inp_hbm.at[bi], tok_v)
        pltpu.sync_copy(spans_hbm.at[bi], span_v)

        def zbody(k, carry):
            base = k * (LANES * LANES)
            for t in range(LANES):
                rows[pl.ds(base + t * LANES, LANES)] = zeros
            return carry

        lax.fori_loop(0, S * VP // (LANES * LANES), zbody, 0, unroll=False)

        lovec = span_v[pl.ds(0, S)]
        hivec = span_v[pl.ds(S, S)]
        los = [lovec[s] for s in range(S)]
        his = [hivec[s] for s in range(S)]
        for off in OFFS:
            tk = tok_v[pl.ds(off, LANES)]
            pos = iota16 + off
            for s in range(S):
                m = (pos >= los[s]) & (pos < his[s])
                plsc.store_scatter(rows, [tk + s * VP], ones, mask=m)

        r0 = bi * S * VP
        pltpu.make_async_copy(rows, bow_hbm.at[pl.ds(r0, S * VP)], sem).start()

    def pair_body(j, carry):
        b0 = base_b + 2 * j
        do_batch(b0, rows0, sem0, j >= 1)
        do_batch(b0 + 1, rows1, sem1, j >= 1)
        return carry

    lax.fori_loop(0, BPW // 2, pair_body, 0, unroll=False)

    pltpu.make_async_copy(rows0, bow_hbm.at[pl.ds(0, S * VP)], sem0).wait()
    pltpu.make_async_copy(rows1, bow_hbm.at[pl.ds(0, S * VP)], sem1).wait()


def _mm_body(bow_ref, wt_ref, b_ref, o_ref):
    o_ref[...] = (
        jnp.dot(bow_ref[...], wt_ref[...], preferred_element_type=jnp.float32)
        + b_ref[...]
    )


def _matmul(bow, wt, b2):
    R = bow.shape[0]
    return pl.pallas_call(
        _mm_body,
        grid=(R // BM,),
        in_specs=[
            pl.BlockSpec((BM, VP), lambda i: (i, 0)),
            pl.BlockSpec((VP, D), lambda i: (0, 0)),
            pl.BlockSpec((1, D), lambda i: (0, 0)),
        ],
        out_specs=pl.BlockSpec((BM, D), lambda i: (i, 0)),
        out_shape=jax.ShapeDtypeStruct((R, D), jnp.float32),
    )(bow, wt, b2)


def kernel(input, span_idxs, W, b):
    spans2 = span_idxs.transpose(0, 2, 1).reshape(B, 2 * S)
    bow = _sc_bow(input, spans2).reshape(B * S, VP)
    wt = jnp.zeros((VP, D), jnp.float32).at[:V].set(W.T)
    out = _matmul(bow, wt, b.reshape(1, D))
    return out.reshape(B, S, D)


# speedup vs baseline: 91.7428x; 1.6285x over previous
import functools

import jax
import jax.numpy as jnp
from jax import lax
from jax.experimental import pallas as pl
from jax.experimental.pallas import tpu as pltpu
from jax.experimental.pallas import tpu_sc as plsc

B, S, L, V, D = 4096, 16, 200, 1000, 128
VP = 1024
BM = 512

NC, NS, LANES = 2, 16, 16
NW = NC * NS
BPW = B // NW
OFFS = list(range(0, L - LANES, LANES)) + [L - LANES]

_sc_mesh = plsc.VectorSubcoreMesh(core_axis_name="c", subcore_axis_name="s")


@functools.partial(
    pl.kernel,
    mesh=_sc_mesh,
    compiler_params=pltpu.CompilerParams(needs_layout_passes=False),
    out_type=jax.ShapeDtypeStruct((B * S, VP), jnp.float32),
    scratch_types=[
        pltpu.VMEM((L,), jnp.int32),
        pltpu.VMEM((2 * S,), jnp.int32),
        pltpu.VMEM((S, VP), jnp.float32),
        pltpu.VMEM((S, VP), jnp.float32),
        pltpu.SemaphoreType.DMA,
        pltpu.SemaphoreType.DMA,
    ],
)
def _sc_bow(inp_hbm, spans_hbm, bow_hbm, tok_v, span_v, rows0, rows1, sem0, sem1):
    ci = lax.axis_index("c")
    si = lax.axis_index("s")
    wid = si * NC + ci
    base_b = wid * BPW

    ones = jnp.ones((LANES,), jnp.float32)
    zeros = jnp.zeros((LANES,), jnp.float32)
    iota16 = lax.iota(jnp.int32, LANES)

    def do_batch(bi, rows, sem, do_wait):
        @pl.when(do_wait)
        def _():
            pltpu.make_async_copy(rows, bow_hbm.at[pl.ds(0, S)], sem).wait()

        pltpu.sync_copy(inp_hbm.at[bi], tok_v)
        pltpu.sync_copy(spans_hbm.at[bi], span_v)

        def zbody(k, carry):
            col = k * LANES
            for s in range(S):
                rows[s, pl.ds(col, LANES)] = zeros
            return carry

        lax.fori_loop(0, VP // LANES, zbody, 0, unroll=False)

        lovec = span_v[pl.ds(0, S)]
        hivec = span_v[pl.ds(S, S)]
        los = [lovec[s] for s in range(S)]
        his = [hivec[s] for s in range(S)]
        for off in OFFS:
            tk = tok_v[pl.ds(off, LANES)]
            pos = iota16 + off
            for s in range(S):
                m = (pos >= los[s]) & (pos < his[s])
                plsc.store_scatter(
                    rows, [jnp.full((LANES,), s, jnp.int32), tk], ones, mask=m
                )

        r0 = bi * S
        pltpu.make_async_copy(rows, bow_hbm.at[pl.ds(r0, S)], sem).start()

    def pair_body(j, carry):
        b0 = base_b + 2 * j
        do_batch(b0, rows0, sem0, j >= 1)
        do_batch(b0 + 1, rows1, sem1, j >= 1)
        return carry

    lax.fori_loop(0, BPW // 2, pair_body, 0, unroll=False)

    pltpu.make_async_copy(rows0, bow_hbm.at[pl.ds(0, S)], sem0).wait()
    pltpu.make_async_copy(rows1, bow_hbm.at[pl.ds(0, S)], sem1).wait()


def _mm_body(bow_ref, wt_ref, b_ref, o_ref):
    o_ref[...] = (
        jnp.dot(bow_ref[...], wt_ref[...], preferred_element_type=jnp.float32)
        + b_ref[...]
    )


def _matmul(bow, wt, b2):
    R = bow.shape[0]
    return pl.pallas_call(
        _mm_body,
        grid=(R // BM,),
        in_specs=[
            pl.BlockSpec((BM, VP), lambda i: (i, 0)),
            pl.BlockSpec((VP, D), lambda i: (0, 0)),
            pl.BlockSpec((1, D), lambda i: (0, 0)),
        ],
        out_specs=pl.BlockSpec((BM, D), lambda i: (i, 0)),
        out_shape=jax.ShapeDtypeStruct((R, D), jnp.float32),
    )(bow, wt, b2)


def kernel(input, span_idxs, W, b):
    spans2 = span_idxs.transpose(0, 2, 1).reshape(B, 2 * S)
    bow = _sc_bow(input, spans2)
    wt = jnp.zeros((VP, D), jnp.float32).at[:V].set(W.T)
    out = _matmul(bow, wt, b.reshape(1, D))
    return out.reshape(B, S, D)


# speedup vs baseline: 96.4771x; 1.0516x over previous
import functools

import jax
import jax.numpy as jnp
from jax import lax
from jax.experimental import pallas as pl
from jax.experimental.pallas import tpu as pltpu
from jax.experimental.pallas import tpu_sc as plsc

B, S, L, V, D = 4096, 16, 200, 1000, 128
VP = 1024
BM = 512

NC, NS, LANES = 2, 16, 16
NW = NC * NS
BPW = B // NW
OFFS = list(range(0, L - LANES, LANES)) + [L - LANES]

_sc_mesh = plsc.VectorSubcoreMesh(core_axis_name="c", subcore_axis_name="s")

NCHUNK = 4
BC = B // NCHUNK
BPWC = BC // NW


@functools.partial(
    pl.kernel,
    mesh=_sc_mesh,
    compiler_params=pltpu.CompilerParams(needs_layout_passes=False),
    out_type=jax.ShapeDtypeStruct((BC * S, VP), jnp.float32),
    scratch_types=[
        pltpu.VMEM((L,), jnp.int32),
        pltpu.VMEM((2 * S,), jnp.int32),
        pltpu.VMEM((S, VP), jnp.float32),
        pltpu.VMEM((S, VP), jnp.float32),
        pltpu.SemaphoreType.DMA,
        pltpu.SemaphoreType.DMA,
    ],
)
def _sc_bow(inp_hbm, spans_hbm, bow_hbm, tok_v, span_v, rows0, rows1, sem0, sem1):
    ci = lax.axis_index("c")
    si = lax.axis_index("s")
    wid = si * NC + ci
    base_b = wid * BPWC

    ones = jnp.ones((LANES,), jnp.float32)
    zeros = jnp.zeros((LANES,), jnp.float32)
    iota16 = lax.iota(jnp.int32, LANES)

    def do_batch(bi, rows, sem, do_wait):
        @pl.when(do_wait)
        def _():
            pltpu.make_async_copy(rows, bow_hbm.at[pl.ds(0, S)], sem).wait()

        pltpu.sync_copy(inp_hbm.at[bi], tok_v)
        pltpu.sync_copy(spans_hbm.at[bi], span_v)

        def zbody(k, carry):
            col = k * LANES
            for s in range(S):
                rows[s, pl.ds(col, LANES)] = zeros
            return carry

        lax.fori_loop(0, VP // LANES, zbody, 0, unroll=False)

        lovec = span_v[pl.ds(0, S)]
        hivec = span_v[pl.ds(S, S)]
        los = [lovec[s] for s in range(S)]
        his = [hivec[s] for s in range(S)]
        for off in OFFS:
            tk = tok_v[pl.ds(off, LANES)]
            pos = iota16 + off
            for s in range(S):
                m = (pos >= los[s]) & (pos < his[s])
                plsc.store_scatter(
                    rows, [jnp.full((LANES,), s, jnp.int32), tk], ones, mask=m
                )

        r0 = bi * S
        pltpu.make_async_copy(rows, bow_hbm.at[pl.ds(r0, S)], sem).start()

    def pair_body(j, carry):
        b0 = base_b + 2 * j
        do_batch(b0, rows0, sem0, j >= 1)
        do_batch(b0 + 1, rows1, sem1, j >= 1)
        return carry

    lax.fori_loop(0, BPWC // 2, pair_body, 0, unroll=False)

    pltpu.make_async_copy(rows0, bow_hbm.at[pl.ds(0, S)], sem0).wait()
    pltpu.make_async_copy(rows1, bow_hbm.at[pl.ds(0, S)], sem1).wait()


def _mm_body(bow_ref, wt_ref, b_ref, o_ref):
    o_ref[...] = (
        jnp.dot(bow_ref[...], wt_ref[...], preferred_element_type=jnp.float32)
        + b_ref[...]
    )


def _matmul(bow, wt, b2):
    R = bow.shape[0]
    return pl.pallas_call(
        _mm_body,
        grid=(R // BM,),
        in_specs=[
            pl.BlockSpec((BM, VP), lambda i: (i, 0)),
            pl.BlockSpec((VP, D), lambda i: (0, 0)),
            pl.BlockSpec((1, D), lambda i: (0, 0)),
        ],
        out_specs=pl.BlockSpec((BM, D), lambda i: (i, 0)),
        out_shape=jax.ShapeDtypeStruct((R, D), jnp.float32),
    )(bow, wt, b2)


def kernel(input, span_idxs, W, b):
    spans2 = span_idxs.transpose(0, 2, 1).reshape(B, 2 * S)
    wt = jnp.zeros((VP, D), jnp.float32).at[:V].set(W.T)
    b2 = b.reshape(1, D)
    outs = []
    for k in range(NCHUNK):
        bow_k = _sc_bow(input[k * BC:(k + 1) * BC], spans2[k * BC:(k + 1) * BC])
        outs.append(_matmul(bow_k, wt, b2))
    return jnp.concatenate(outs, axis=0).reshape(B, S, D)


# speedup vs baseline: 124.9550x; 1.2952x over previous
import functools

import jax
import jax.numpy as jnp
from jax import lax
from jax.experimental import pallas as pl
from jax.experimental.pallas import tpu as pltpu
from jax.experimental.pallas import tpu_sc as plsc

B, S, L, V, D = 4096, 16, 200, 1000, 128
VP = 1024
BM = 512

NC, NS, LANES = 2, 16, 16
NW = NC * NS
BPW = B // NW
OFFS = list(range(0, L - LANES, LANES)) + [L - LANES]

_sc_mesh = plsc.VectorSubcoreMesh(core_axis_name="c", subcore_axis_name="s")

NCHUNK = 4
BC = B // NCHUNK
BPWC = BC // NW
QUAD = 4


@functools.partial(
    pl.kernel,
    mesh=_sc_mesh,
    compiler_params=pltpu.CompilerParams(needs_layout_passes=False),
    out_type=jax.ShapeDtypeStruct((BC * S, VP), jnp.float32),
    scratch_types=[
        pltpu.VMEM((QUAD * L,), jnp.int32),
        pltpu.VMEM((QUAD * 2 * S,), jnp.int32),
        pltpu.VMEM((S, VP), jnp.float32),
        pltpu.VMEM((S, VP), jnp.float32),
        pltpu.SemaphoreType.DMA,
        pltpu.SemaphoreType.DMA,
        pltpu.SemaphoreType.DMA,
    ],
)
def _sc_bow(inp_hbm, spans_hbm, bow_hbm, tok_v, span_v, r0_v, r1_v,
            fsem, osem0, osem1):
    ci = lax.axis_index("c")
    si = lax.axis_index("s")
    wid = si * NC + ci
    base_b = wid * BPWC

    rows = [r0_v, r1_v]
    osems = [osem0, osem1]
    ones = jnp.ones((LANES,), jnp.float32)
    zeros = jnp.zeros((LANES,), jnp.float32)
    iota16 = lax.iota(jnp.int32, LANES)

    def quad_body(j, carry):
        qb = base_b + j * QUAD
        tok_cp = pltpu.make_async_copy(
            inp_hbm.at[pl.ds(qb * L, QUAD * L)], tok_v, fsem)
        span_cp = pltpu.make_async_copy(
            spans_hbm.at[pl.ds(qb * 2 * S, QUAD * 2 * S)], span_v, fsem)
        tok_cp.start()
        span_cp.start()
        tok_cp.wait()
        span_cp.wait()

        for q in range(QUAD):
            p = q % 2
            buf = rows[p]
            t_ge2 = (q >= 2) or None

            def recycle():
                pltpu.make_async_copy(
                    buf, bow_hbm.at[pl.ds(0, S)], osems[p]).wait()

            if q >= 2:
                recycle()
            else:
                pl.when(j >= 1)(recycle)

            def zbody(k, carry2):
                col = k * LANES
                for s in range(S):
                    buf[s, pl.ds(col, LANES)] = zeros
                return carry2

            lax.fori_loop(0, VP // LANES, zbody, 0, unroll=False)

            lovec = span_v[pl.ds(q * 2 * S, S)]
            hivec = span_v[pl.ds(q * 2 * S + S, S)]
            los = [lovec[s] for s in range(S)]
            his = [hivec[s] for s in range(S)]
            for off in OFFS:
                tk = tok_v[pl.ds(q * L + off, LANES)]
                pos = iota16 + off
                for s in range(S):
                    m = (pos >= los[s]) & (pos < his[s])
                    plsc.store_scatter(
                        buf, [jnp.full((LANES,), s, jnp.int32), tk],
                        ones, mask=m)

            rr = (qb + q) * S
            pltpu.make_async_copy(
                buf, bow_hbm.at[pl.ds(rr, S)], osems[p]).start()
        return carry

    lax.fori_loop(0, BPWC // QUAD, quad_body, 0, unroll=False)

    pltpu.make_async_copy(r0_v, bow_hbm.at[pl.ds(0, S)], osem0).wait()
    pltpu.make_async_copy(r1_v, bow_hbm.at[pl.ds(0, S)], osem1).wait()


def _mm_body(bow_ref, wt_ref, b_ref, o_ref):
    o_ref[...] = (
        jnp.dot(bow_ref[...], wt_ref[...], preferred_element_type=jnp.float32)
        + b_ref[...]
    )


def _matmul(bow, wt, b2):
    R = bow.shape[0]
    return pl.pallas_call(
        _mm_body,
        grid=(R // BM,),
        in_specs=[
            pl.BlockSpec((BM, VP), lambda i: (i, 0)),
            pl.BlockSpec((VP, D), lambda i: (0, 0)),
            pl.BlockSpec((1, D), lambda i: (0, 0)),
        ],
        out_specs=pl.BlockSpec((BM, D), lambda i: (i, 0)),
        out_shape=jax.ShapeDtypeStruct((R, D), jnp.float32),
    )(bow, wt, b2)


def kernel(input, span_idxs, W, b):
    spans2 = span_idxs.transpose(0, 2, 1).reshape(B, 2 * S)
    wt = jnp.zeros((VP, D), jnp.float32).at[:V].set(W.T)
    b2 = b.reshape(1, D)
    outs = []
    inp_flat = input.reshape(B * L)
    spans_flat = spans2.reshape(B * 2 * S)
    for k in range(NCHUNK):
        bow_k = _sc_bow(
            inp_flat[k * BC * L:(k + 1) * BC * L],
            spans_flat[k * BC * 2 * S:(k + 1) * BC * 2 * S],
        )
        outs.append(_matmul(bow_k, wt, b2))
    return jnp.concatenate(outs, axis=0).reshape(B, S, D)


# speedup vs baseline: 125.3043x; 1.0028x over previous
import functools

import jax
import jax.numpy as jnp
from jax import lax
from jax.experimental import pallas as pl
from jax.experimental.pallas import tpu as pltpu
from jax.experimental.pallas import tpu_sc as plsc

B, S, L, V, D = 4096, 16, 200, 1000, 128
VP = 1024
BM = 1024

NC, NS, LANES = 2, 16, 16
NW = NC * NS
BPW = B // NW
OFFS = list(range(0, L - LANES, LANES)) + [L - LANES]

_sc_mesh = plsc.VectorSubcoreMesh(core_axis_name="c", subcore_axis_name="s")

NCHUNK = 4
BC = B // NCHUNK
BPWC = BC // NW
QUAD = 4


@functools.partial(
    pl.kernel,
    mesh=_sc_mesh,
    compiler_params=pltpu.CompilerParams(needs_layout_passes=False),
    out_type=jax.ShapeDtypeStruct((BC * S, VP), jnp.float32),
    scratch_types=[
        pltpu.VMEM((QUAD * L,), jnp.int32),
        pltpu.VMEM((QUAD * 2 * S,), jnp.int32),
        pltpu.VMEM((S, VP), jnp.float32),
        pltpu.VMEM((S, VP), jnp.float32),
        pltpu.SemaphoreType.DMA,
        pltpu.SemaphoreType.DMA,
        pltpu.SemaphoreType.DMA,
    ],
)
def _sc_bow(inp_hbm, spans_hbm, bow_hbm, tok_v, span_v, r0_v, r1_v,
            fsem, osem0, osem1):
    ci = lax.axis_index("c")
    si = lax.axis_index("s")
    wid = si * NC + ci
    base_b = wid * BPWC

    rows = [r0_v, r1_v]
    osems = [osem0, osem1]
    ones = jnp.ones((LANES,), jnp.float32)
    zeros = jnp.zeros((LANES,), jnp.float32)
    iota16 = lax.iota(jnp.int32, LANES)

    def quad_body(j, carry):
        qb = base_b + j * QUAD
        tok_cp = pltpu.make_async_copy(
            inp_hbm.at[pl.ds(qb * L, QUAD * L)], tok_v, fsem)
        span_cp = pltpu.make_async_copy(
            spans_hbm.at[pl.ds(qb * 2 * S, QUAD * 2 * S)], span_v, fsem)
        tok_cp.start()
        span_cp.start()
        tok_cp.wait()
        span_cp.wait()

        for q in range(QUAD):
            p = q % 2
            buf = rows[p]
            t_ge2 = (q >= 2) or None

            def recycle():
                pltpu.make_async_copy(
                    buf, bow_hbm.at[pl.ds(0, S)], osems[p]).wait()

            if q >= 2:
                recycle()
            else:
                pl.when(j >= 1)(recycle)

            def zbody(k, carry2):
                col = k * (2 * LANES)
                for s in range(S):
                    buf[s, pl.ds(col, LANES)] = zeros
                    buf[s, pl.ds(col + LANES, LANES)] = zeros
                return carry2

            lax.fori_loop(0, VP // (2 * LANES), zbody, 0, unroll=False)

            lovec = span_v[pl.ds(q * 2 * S, S)]
            hivec = span_v[pl.ds(q * 2 * S + S, S)]
            los = [lovec[s] for s in range(S)]
            his = [hivec[s] for s in range(S)]
            for off in OFFS:
                tk = tok_v[pl.ds(q * L + off, LANES)]
                pos = iota16 + off
                for s in range(S):
                    m = (pos >= los[s]) & (pos < his[s])
                    plsc.store_scatter(
                        buf, [jnp.full((LANES,), s, jnp.int32), tk],
                        ones, mask=m)

            rr = (qb + q) * S
            pltpu.make_async_copy(
                buf, bow_hbm.at[pl.ds(rr, S)], osems[p]).start()
        return carry

    lax.fori_loop(0, BPWC // QUAD, quad_body, 0, unroll=False)

    pltpu.make_async_copy(r0_v, bow_hbm.at[pl.ds(0, S)], osem0).wait()
    pltpu.make_async_copy(r1_v, bow_hbm.at[pl.ds(0, S)], osem1).wait()


def _mm_body(bow_ref, wt_ref, b_ref, o_ref):
    o_ref[...] = (
        jnp.dot(bow_ref[...], wt_ref[...], preferred_element_type=jnp.float32)
        + b_ref[...]
    )


def _matmul(bow, wt, b2):
    R = bow.shape[0]
    return pl.pallas_call(
        _mm_body,
        grid=(R // BM,),
        in_specs=[
            pl.BlockSpec((BM, VP), lambda i: (i, 0)),
            pl.BlockSpec((VP, D), lambda i: (0, 0)),
            pl.BlockSpec((1, D), lambda i: (0, 0)),
        ],
        out_specs=pl.BlockSpec((BM, D), lambda i: (i, 0)),
        out_shape=jax.ShapeDtypeStruct((R, D), jnp.float32),
    )(bow, wt, b2)


def kernel(input, span_idxs, W, b):
    spans2 = span_idxs.transpose(0, 2, 1).reshape(B, 2 * S)
    wt = jnp.zeros((VP, D), jnp.float32).at[:V].set(W.T)
    b2 = b.reshape(1, D)
    outs = []
    inp_flat = input.reshape(B * L)
    spans_flat = spans2.reshape(B * 2 * S)
    for k in range(NCHUNK):
        bow_k = _sc_bow(
            inp_flat[k * BC * L:(k + 1) * BC * L],
            spans_flat[k * BC * 2 * S:(k + 1) * BC * 2 * S],
        )
        outs.append(_matmul(bow_k, wt, b2))
    return jnp.concatenate(outs, axis=0).reshape(B, S, D)
